# baseline (speedup 1.0000x reference)
import functools

import jax
import jax.numpy as jnp
from jax import lax
from jax.experimental import pallas as pl
from jax.experimental.pallas import tpu as pltpu
from jax.experimental.pallas import tpu_sc as plsc

N = 10000
E = 320000
D = 128
NP = 10240
EP = 327680
NT = 16
NC = 2
NW = NT * NC
CH = 1024
SW = 32
CR = CH // SW
NCH = EP // NW // CH
NPT = NP // NT
DR = NP // 128


def _tc_pre_body(x_ref, wm_ref, a12_ref, b12_ref, kw_ref, val_ref, s12_ref):
    xb = x_ref[...]
    val_ref[...] = jnp.dot(xb, kw_ref[...], preferred_element_type=jnp.float32)
    w12 = jnp.dot(wm_ref[...], a12_ref[...], preferred_element_type=jnp.float32)
    s12_ref[...] = (jnp.dot(xb, w12, preferred_element_type=jnp.float32)
                    + b12_ref[...])


def _tc_precompute(x, W_map, a12, b12, kw):
    bn = 1000
    grid = N // bn
    return pl.pallas_call(
        _tc_pre_body,
        grid=(grid,),
        in_specs=[
            pl.BlockSpec((bn, D), lambda i: (i, 0)),
            pl.BlockSpec((D, D), lambda i: (0, 0)),
            pl.BlockSpec((D, 2), lambda i: (0, 0)),
            pl.BlockSpec((1, 2), lambda i: (0, 0)),
            pl.BlockSpec((D, D), lambda i: (0, 0)),
        ],
        out_specs=[
            pl.BlockSpec((bn, D), lambda i: (i, 0)),
            pl.BlockSpec((bn, 2), lambda i: (i, 0)),
        ],
        out_shape=[
            jax.ShapeDtypeStruct((N, D), jnp.float32),
            jax.ShapeDtypeStruct((N, 2), jnp.float32),
        ],
    )(x, W_map, a12, b12, kw)


def _tc_comb_body(part_ref, dn_ref, bias_ref, out_ref):
    dsum = jnp.sum(dn_ref[...], axis=1, keepdims=True)
    dsum = jnp.where(dsum > 0.0, dsum, 1.0)
    acc = part_ref[0] + part_ref[1]
    out_ref[...] = acc * (1.0 / dsum) + bias_ref[...]


def _tc_combine(part, dnT, bias):
    bn = 1000
    grid = N // bn
    return pl.pallas_call(
        _tc_comb_body,
        grid=(grid,),
        in_specs=[
            pl.BlockSpec((NC, bn, D), lambda i: (0, i, 0)),
            pl.BlockSpec((bn, NW), lambda i: (i, 0)),
            pl.BlockSpec((bn, D), lambda i: (i, 0)),
        ],
        out_specs=pl.BlockSpec((bn, D), lambda i: (i, 0)),
        out_shape=jax.ShapeDtypeStruct((N, D), jnp.float32),
    )(part, dnT, bias)


def _sc_body(rows_hbm, cols_hbm, s1_hbm, s2_hbm, value_hbm,
             part_hbm, dn_hbm,
             s1_v, s2_v, denom_v, ri, ci, p_v,
             vb0, vb1, sg0, sg1, ss0, ss1, si, out_sh):
    c = lax.axis_index("c")
    s = lax.axis_index("s")
    wid = c * NT + s

    pltpu.sync_copy(s1_hbm, s1_v)
    pltpu.sync_copy(s2_hbm, s2_v)

    zero16 = jnp.zeros((16,), jnp.float32)

    def _zd(r, _):
        for l in range(8):
            denom_v[r, pl.ds(l * 16, 16)] = zero16
        return 0
    lax.fori_loop(0, DR, _zd, 0)

    slots = ((vb0, sg0, ss0), (vb1, sg1, ss1))

    def _zv(r, _):
        for vb in (vb0, vb1):
            for l in range(8):
                vb[r, pl.ds(l * 16, 16)] = zero16
        return 0
    lax.fori_loop(0, SW, _zv, 0)
    for k in range(NPT // (2 * SW)):
        for i, (vb, sg, _ss) in enumerate(slots):
            pltpu.async_copy(
                vb, out_sh.at[pl.ds(s * NPT + (k * 2 + i) * SW, SW)], sg)
        for vb, sg, _ss in slots:
            pltpu.make_async_copy(vb, out_sh.at[pl.ds(0, SW)], sg).wait()
    plsc.subcore_barrier()

    def _load_idx(edge0, h):
        pltpu.async_copy(rows_hbm.at[pl.ds(edge0, CH)], ri.at[h], si)
        pltpu.async_copy(cols_hbm.at[pl.ds(edge0, CH)], ci.at[h], si)

    def _wait_idx():
        pltpu.make_async_copy(rows_hbm.at[pl.ds(0, CH)], ri.at[0], si).wait()
        pltpu.make_async_copy(cols_hbm.at[pl.ds(0, CH)], ci.at[0], si).wait()

    def _gissue(h, k, slot):
        pltpu.async_copy(value_hbm.at[ci.at[h, pl.ds(k * SW, SW)]],
                         slots[slot][0], slots[slot][1])

    base = wid * NCH * CH
    _load_idx(base, 0)

    def _chunk(ch, _):
        _wait_idx()
        h = ch % 2

        @pl.when(ch > 0)
        def _drain_prev():
            ix0 = ri.at[h, pl.ds(0, SW)]
            for vb_t, _sg_t, ss_t in slots:
                pltpu.make_async_copy(vb_t, out_sh.at[ix0], ss_t).wait()

        @pl.when(ch + 1 < NCH)
        def _pref():
            _load_idx(base + (ch + 1) * CH, (ch + 1) % 2)

        _gissue(h, 0, 0)

        def _step(k, _):
            rix = ri.at[h, pl.ds(k * SW, SW)]
            for l in range(SW // 16):
                rv = ri[h, pl.ds(k * SW + l * 16, 16)]
                cv = ci[h, pl.ds(k * SW + l * 16, 16)]
                p = _edge_p(s1_v, s2_v, rv, cv)
                plsc.addupdate_scatter(
                    denom_v,
                    [lax.shift_right_logical(rv, 7),
                     lax.bitwise_and(rv, 127)],
                    p)
                p_v[pl.ds(l * 16, 16)] = p

            for sl in range(2):
                vb_c, sg_c, ss_c = slots[sl]
                vb_n, sg_n, ss_n = slots[1 - sl]

                @pl.when(k % 2 == sl)
                def _proc():
                    pltpu.make_async_copy(
                        value_hbm.at[rix], vb_c, sg_c).wait()

                    def _scale(j2, _):
                        aj = plsc.load_gather(
                            p_v, [jnp.full((16,), j2, jnp.int32)])
                        for l in range(8):
                            vb_c[j2, pl.ds(l * 16, 16)] = (
                                vb_c[j2, pl.ds(l * 16, 16)] * aj)
                        return 0
                    lax.fori_loop(0, SW, _scale, 0)
                    pltpu.async_copy(vb_c, out_sh.at[rix], ss_c, add=True)

                    @pl.when((k >= 1) & (k + 1 < CR))
                    def _w():
                        pltpu.make_async_copy(
                            vb_n, out_sh.at[rix], ss_n).wait()

                    @pl.when(k + 1 < CR)
                    def _g():
                        _gissue(h, k + 1, 1 - sl)
            return 0
        lax.fori_loop(0, CR, _step, 0)
        return 0
    lax.fori_loop(0, NCH, _chunk, 0)

    ix0 = ri.at[(NCH - 1) % 2, pl.ds(0, SW)]
    for vb_t, _sg_t, ss_t in slots:
        pltpu.make_async_copy(vb_t, out_sh.at[ix0], ss_t).wait()
    plsc.subcore_barrier()

    pltpu.sync_copy(out_sh.at[pl.ds(s * NPT, NPT)],
                    part_hbm.at[c, pl.ds(s * NPT, NPT)])
    pltpu.sync_copy(denom_v, dn_hbm.at[c, s])


def _edge_p(s1_v, s2_v, rv, cv):
    v1 = plsc.load_gather(s1_v, [rv])
    v2 = plsc.load_gather(s2_v, [cv])
    e = v1 + v2
    e = jnp.where(e >= 0.0, e, 0.2 * e)
    return jnp.exp(e)


def _sc_edge_kernel(rows1d, cols1d, s1p, s2p, value):
    mesh = plsc.VectorSubcoreMesh(core_axis_name="c", subcore_axis_name="s")
    f = functools.partial(
        pl.kernel,
        mesh=mesh,
        compiler_params=pltpu.CompilerParams(needs_layout_passes=False),
        out_type=[
            jax.ShapeDtypeStruct((NC, NP, D), jnp.float32),
            jax.ShapeDtypeStruct((NC, NT, DR, 128), jnp.float32),
        ],
        scratch_types=[
            pltpu.VMEM((NP,), jnp.float32),
            pltpu.VMEM((NP,), jnp.float32),
            pltpu.VMEM((DR, 128), jnp.float32),
            pltpu.VMEM((2, CH), jnp.int32),
            pltpu.VMEM((2, CH), jnp.int32),
            pltpu.VMEM((SW,), jnp.float32),
            pltpu.VMEM((SW, D), jnp.float32),
            pltpu.VMEM((SW, D), jnp.float32),
            pltpu.SemaphoreType.DMA,
            pltpu.SemaphoreType.DMA,
            pltpu.SemaphoreType.DMA,
            pltpu.SemaphoreType.DMA,
            pltpu.SemaphoreType.DMA,
            pltpu.VMEM_SHARED((NP, D), jnp.float32),
        ],
    )(_sc_body)
    return f(rows1d, cols1d, s1p, s2p, value)


def kernel(x, edge_index, W_map, a1, b1, a2, b2, kernel, bias):
    a12 = jnp.concatenate([a1, a2], axis=1)
    b12 = jnp.stack([b1[0], b2[0]]).reshape(1, 2)
    value, s12 = _tc_precompute(x, W_map, a12, b12, kernel)

    npad = EP - E
    ar = jnp.arange(npad, dtype=jnp.int32)
    prow = N + (ar % (NP - N))
    pcol = ar % 9973
    rows = jnp.concatenate([edge_index[0], prow])
    cols = jnp.concatenate([edge_index[1], pcol])

    zpad = jnp.zeros((NP - N,), jnp.float32)
    s1p = jnp.concatenate([s12[:, 0], zpad])
    s2p = jnp.concatenate([s12[:, 1], zpad])

    part, dn = _sc_edge_kernel(rows, cols, s1p, s2p, value)
    dnT = jnp.transpose(dn.reshape(NW, NP))
    return _tc_combine(part, dnT, bias)


# speedup vs baseline: 1.6002x; 1.6002x over previous
import functools

import jax
import jax.numpy as jnp
from jax import lax
from jax.experimental import pallas as pl
from jax.experimental.pallas import tpu as pltpu
from jax.experimental.pallas import tpu_sc as plsc

N = 10000
E = 320000
D = 128
NP = 10240
EP = 327680
NT = 16
NC = 2
NW = NT * NC
CH = 1024
SW = 32
NS = 3
CR = CH // SW
NCH = EP // NW // CH
NPT = NP // NT
DR = NP // 128


def _tc_pre_body(x_ref, wm_ref, a12_ref, b12_ref, kw_ref, val_ref, s12_ref):
    xb = x_ref[...]
    val_ref[...] = jnp.dot(xb, kw_ref[...], preferred_element_type=jnp.float32)
    w12 = jnp.dot(wm_ref[...], a12_ref[...], preferred_element_type=jnp.float32)
    s12_ref[...] = (jnp.dot(xb, w12, preferred_element_type=jnp.float32)
                    + b12_ref[...])


def _tc_precompute(x, W_map, a12, b12, kw):
    bn = 1000
    grid = N // bn
    return pl.pallas_call(
        _tc_pre_body,
        grid=(grid,),
        in_specs=[
            pl.BlockSpec((bn, D), lambda i: (i, 0)),
            pl.BlockSpec((D, D), lambda i: (0, 0)),
            pl.BlockSpec((D, 2), lambda i: (0, 0)),
            pl.BlockSpec((1, 2), lambda i: (0, 0)),
            pl.BlockSpec((D, D), lambda i: (0, 0)),
        ],
        out_specs=[
            pl.BlockSpec((bn, D), lambda i: (i, 0)),
            pl.BlockSpec((bn, 2), lambda i: (i, 0)),
        ],
        out_shape=[
            jax.ShapeDtypeStruct((N, D), jnp.float32),
            jax.ShapeDtypeStruct((N, 2), jnp.float32),
        ],
    )(x, W_map, a12, b12, kw)


def _tc_comb_body(part_ref, dn_ref, bias_ref, out_ref):
    dsum = jnp.sum(dn_ref[...], axis=1, keepdims=True)
    dsum = jnp.where(dsum > 0.0, dsum, 1.0)
    acc = part_ref[0] + part_ref[1]
    out_ref[...] = acc * (1.0 / dsum) + bias_ref[...]


def _tc_combine(part, dnT, bias):
    bn = 1000
    grid = N // bn
    return pl.pallas_call(
        _tc_comb_body,
        grid=(grid,),
        in_specs=[
            pl.BlockSpec((NC, bn, D), lambda i: (0, i, 0)),
            pl.BlockSpec((bn, NW), lambda i: (i, 0)),
            pl.BlockSpec((bn, D), lambda i: (i, 0)),
        ],
        out_specs=pl.BlockSpec((bn, D), lambda i: (i, 0)),
        out_shape=jax.ShapeDtypeStruct((N, D), jnp.float32),
    )(part, dnT, bias)


def _sc_body(rows_hbm, cols_hbm, s1_hbm, s2_hbm, value_hbm,
             part_hbm, dn_hbm,
             s1_v, s2_v, denom_v, ri, ci, p_v,
             vb0, vb1, vb2, sg0, sg1, sg2, ss0, ss1, ss2, si, out_sh):
    c = lax.axis_index("c")
    s = lax.axis_index("s")
    wid = c * NT + s

    pltpu.sync_copy(s1_hbm, s1_v)
    pltpu.sync_copy(s2_hbm, s2_v)

    zero16 = jnp.zeros((16,), jnp.float32)

    def _zd(r, _):
        for l in range(8):
            denom_v[r, pl.ds(l * 16, 16)] = zero16
        return 0
    lax.fori_loop(0, DR, _zd, 0)

    slots = ((vb0, sg0, ss0), (vb1, sg1, ss1), (vb2, sg2, ss2))

    def _zv(r, _):
        for vb in (vb0, vb1, vb2):
            for l in range(8):
                vb[r, pl.ds(l * 16, 16)] = zero16
        return 0
    lax.fori_loop(0, SW, _zv, 0)
    for k in range(NPT // (4 * SW)):
        for i in range(4):
            vb, sg, _ss = slots[i % NS]
            if i == NS:
                pltpu.make_async_copy(vb, out_sh.at[pl.ds(0, SW)], sg).wait()
            pltpu.async_copy(
                vb, out_sh.at[pl.ds(s * NPT + (k * 4 + i) * SW, SW)], sg)
        for i in range(NS):
            vb, sg, _ss = slots[i]
            pltpu.make_async_copy(vb, out_sh.at[pl.ds(0, SW)], sg).wait()
    plsc.subcore_barrier()

    def _load_idx(edge0, h):
        pltpu.async_copy(rows_hbm.at[pl.ds(edge0, CH)], ri.at[h], si)
        pltpu.async_copy(cols_hbm.at[pl.ds(edge0, CH)], ci.at[h], si)

    def _wait_idx():
        pltpu.make_async_copy(rows_hbm.at[pl.ds(0, CH)], ri.at[0], si).wait()
        pltpu.make_async_copy(cols_hbm.at[pl.ds(0, CH)], ci.at[0], si).wait()

    def _gissue(h, k, slot):
        pltpu.async_copy(value_hbm.at[ci.at[h, pl.ds(k * SW, SW)]],
                         slots[slot][0], slots[slot][1])

    base = wid * NCH * CH
    _load_idx(base, 0)

    def _chunk(ch, _):
        _wait_idx()
        h = ch % 2

        @pl.when(ch > 0)
        def _drain_prev():
            ix0 = ri.at[h, pl.ds(0, SW)]
            for vb_t, _sg_t, ss_t in slots:
                pltpu.make_async_copy(vb_t, out_sh.at[ix0], ss_t).wait()

        @pl.when(ch + 1 < NCH)
        def _pref():
            _load_idx(base + (ch + 1) * CH, (ch + 1) % 2)

        _gissue(h, 0, 0)
        _gissue(h, 1, 1)

        def _step(k, _):
            rix = ri.at[h, pl.ds(k * SW, SW)]
            for l in range(SW // 16):
                rv = ri[h, pl.ds(k * SW + l * 16, 16)]
                cv = ci[h, pl.ds(k * SW + l * 16, 16)]
                p = _edge_p(s1_v, s2_v, rv, cv)
                plsc.addupdate_scatter(
                    denom_v,
                    [lax.shift_right_logical(rv, 7),
                     lax.bitwise_and(rv, 127)],
                    p)
                p_v[pl.ds(l * 16, 16)] = p

            for sl in range(NS):
                vb_c, sg_c, ss_c = slots[sl]
                nsl = (sl + 2) % NS
                vb_n, sg_n, ss_n = slots[nsl]

                @pl.when(k % NS == sl)
                def _proc():
                    pltpu.make_async_copy(
                        value_hbm.at[rix], vb_c, sg_c).wait()

                    def _scale(j2, _):
                        aj = plsc.load_gather(
                            p_v, [jnp.full((16,), j2, jnp.int32)])
                        for l in range(8):
                            vb_c[j2, pl.ds(l * 16, 16)] = (
                                vb_c[j2, pl.ds(l * 16, 16)] * aj)
                        return 0
                    lax.fori_loop(0, SW, _scale, 0)
                    pltpu.async_copy(vb_c, out_sh.at[rix], ss_c, add=True)

                    @pl.when((k >= 1) & (k + 2 < CR))
                    def _w():
                        pltpu.make_async_copy(
                            vb_n, out_sh.at[rix], ss_n).wait()

                    @pl.when(k + 2 < CR)
                    def _g():
                        _gissue(h, k + 2, nsl)
            return 0
        lax.fori_loop(0, CR, _step, 0)
        return 0
    lax.fori_loop(0, NCH, _chunk, 0)

    ix0 = ri.at[(NCH - 1) % 2, pl.ds(0, SW)]
    for vb_t, _sg_t, ss_t in slots:
        pltpu.make_async_copy(vb_t, out_sh.at[ix0], ss_t).wait()
    plsc.subcore_barrier()

    pltpu.sync_copy(out_sh.at[pl.ds(s * NPT, NPT)],
                    part_hbm.at[c, pl.ds(s * NPT, NPT)])
    pltpu.sync_copy(denom_v, dn_hbm.at[c, s])


def _edge_p(s1_v, s2_v, rv, cv):
    v1 = plsc.load_gather(s1_v, [rv])
    v2 = plsc.load_gather(s2_v, [cv])
    e = v1 + v2
    e = jnp.where(e >= 0.0, e, 0.2 * e)
    return jnp.exp(e)


def _sc_edge_kernel(rows1d, cols1d, s1p, s2p, value):
    mesh = plsc.VectorSubcoreMesh(core_axis_name="c", subcore_axis_name="s")
    f = functools.partial(
        pl.kernel,
        mesh=mesh,
        compiler_params=pltpu.CompilerParams(needs_layout_passes=False),
        out_type=[
            jax.ShapeDtypeStruct((NC, NP, D), jnp.float32),
            jax.ShapeDtypeStruct((NC, NT, DR, 128), jnp.float32),
        ],
        scratch_types=[
            pltpu.VMEM((NP,), jnp.float32),
            pltpu.VMEM((NP,), jnp.float32),
            pltpu.VMEM((DR, 128), jnp.float32),
            pltpu.VMEM((2, CH), jnp.int32),
            pltpu.VMEM((2, CH), jnp.int32),
            pltpu.VMEM((SW,), jnp.float32),
            pltpu.VMEM((SW, D), jnp.float32),
            pltpu.VMEM((SW, D), jnp.float32),
            pltpu.VMEM((SW, D), jnp.float32),
            pltpu.SemaphoreType.DMA,
            pltpu.SemaphoreType.DMA,
            pltpu.SemaphoreType.DMA,
            pltpu.SemaphoreType.DMA,
            pltpu.SemaphoreType.DMA,
            pltpu.SemaphoreType.DMA,
            pltpu.SemaphoreType.DMA,
            pltpu.VMEM_SHARED((NP, D), jnp.float32),
        ],
    )(_sc_body)
    return f(rows1d, cols1d, s1p, s2p, value)


def kernel(x, edge_index, W_map, a1, b1, a2, b2, kernel, bias):
    a12 = jnp.concatenate([a1, a2], axis=1)
    b12 = jnp.stack([b1[0], b2[0]]).reshape(1, 2)
    value, s12 = _tc_precompute(x, W_map, a12, b12, kernel)

    npad = EP - E
    ar = jnp.arange(npad, dtype=jnp.int32)
    prow = N + (ar % (NP - N))
    pcol = ar % 9973
    rows = jnp.concatenate([edge_index[0], prow])
    cols = jnp.concatenate([edge_index[1], pcol])

    zpad = jnp.zeros((NP - N,), jnp.float32)
    s1p = jnp.concatenate([s12[:, 0], zpad])
    s2p = jnp.concatenate([s12[:, 1], zpad])

    part, dn = _sc_edge_kernel(rows, cols, s1p, s2p, value)
    dnT = jnp.transpose(dn.reshape(NW, NP))
    return _tc_combine(part, dnT, bias)


# speedup vs baseline: 1.8371x; 1.1481x over previous
import functools

import jax
import jax.numpy as jnp
from jax import lax
from jax.experimental import pallas as pl
from jax.experimental.pallas import tpu as pltpu
from jax.experimental.pallas import tpu_sc as plsc

N = 10000
E = 320000
D = 128
NP = 10240
EP = 327680
NT = 16
NC = 2
NW = NT * NC
CH = 1024
SW = 32
NS = 3
CR = CH // SW
NCH = EP // NW // CH
NPT = NP // NT
DR = NP // 128


def _tc_pre_body(x_ref, wm_ref, a12_ref, b12_ref, kw_ref, val_ref, s12_ref):
    xb = x_ref[...]
    val_ref[...] = jnp.dot(xb, kw_ref[...], preferred_element_type=jnp.float32)
    w12 = jnp.dot(wm_ref[...], a12_ref[...], preferred_element_type=jnp.float32)
    s12_ref[...] = (jnp.dot(xb, w12, preferred_element_type=jnp.float32)
                    + b12_ref[...])


def _tc_precompute(x, W_map, a12, b12, kw):
    bn = 1000
    grid = N // bn
    return pl.pallas_call(
        _tc_pre_body,
        grid=(grid,),
        in_specs=[
            pl.BlockSpec((bn, D), lambda i: (i, 0)),
            pl.BlockSpec((D, D), lambda i: (0, 0)),
            pl.BlockSpec((D, 2), lambda i: (0, 0)),
            pl.BlockSpec((1, 2), lambda i: (0, 0)),
            pl.BlockSpec((D, D), lambda i: (0, 0)),
        ],
        out_specs=[
            pl.BlockSpec((bn, D), lambda i: (i, 0)),
            pl.BlockSpec((bn, 2), lambda i: (i, 0)),
        ],
        out_shape=[
            jax.ShapeDtypeStruct((N, D), jnp.float32),
            jax.ShapeDtypeStruct((N, 2), jnp.float32),
        ],
    )(x, W_map, a12, b12, kw)


def _tc_comb_body(part_ref, dn_ref, bias_ref, out_ref):
    dsum = jnp.sum(dn_ref[...], axis=1, keepdims=True)
    dsum = jnp.where(dsum > 0.0, dsum, 1.0)
    acc = part_ref[0] + part_ref[1]
    out_ref[...] = acc * (1.0 / dsum) + bias_ref[...]


def _tc_combine(part, dnT, bias):
    bn = 1000
    grid = N // bn
    return pl.pallas_call(
        _tc_comb_body,
        grid=(grid,),
        in_specs=[
            pl.BlockSpec((NC, bn, D), lambda i: (0, i, 0)),
            pl.BlockSpec((bn, NW), lambda i: (i, 0)),
            pl.BlockSpec((bn, D), lambda i: (i, 0)),
        ],
        out_specs=pl.BlockSpec((bn, D), lambda i: (i, 0)),
        out_shape=jax.ShapeDtypeStruct((N, D), jnp.float32),
    )(part, dnT, bias)


def _sc_body(rows_hbm, cols_hbm, s1_hbm, s2_hbm, value_hbm,
             part_hbm, dn_hbm,
             s1_v, s2_v, denom_v, ri, ci, p_v,
             vb0, vb1, vb2, sg0, sg1, sg2, ss0, ss1, ss2, si, out_sh):
    c = lax.axis_index("c")
    s = lax.axis_index("s")
    wid = c * NT + s

    pltpu.sync_copy(s1_hbm, s1_v)
    pltpu.sync_copy(s2_hbm, s2_v)

    zero16 = jnp.zeros((16,), jnp.float32)

    def _zd(r, _):
        for l in range(8):
            denom_v[r, pl.ds(l * 16, 16)] = zero16
        return 0
    lax.fori_loop(0, DR, _zd, 0)

    slots = ((vb0, sg0, ss0), (vb1, sg1, ss1), (vb2, sg2, ss2))

    def _zv(r, _):
        for vb in (vb0, vb1, vb2):
            for l in range(8):
                vb[r, pl.ds(l * 16, 16)] = zero16
        return 0
    lax.fori_loop(0, SW, _zv, 0)
    for k in range(NPT // (4 * SW)):
        for i in range(4):
            vb, sg, _ss = slots[i % NS]
            if i == NS:
                pltpu.make_async_copy(vb, out_sh.at[pl.ds(0, SW)], sg).wait()
            pltpu.async_copy(
                vb, out_sh.at[pl.ds(s * NPT + (k * 4 + i) * SW, SW)], sg)
        for i in range(NS):
            vb, sg, _ss = slots[i]
            pltpu.make_async_copy(vb, out_sh.at[pl.ds(0, SW)], sg).wait()
    plsc.subcore_barrier()

    def _load_idx(edge0, h):
        pltpu.async_copy(rows_hbm.at[pl.ds(edge0, CH)], ri.at[h], si)
        pltpu.async_copy(cols_hbm.at[pl.ds(edge0, CH)], ci.at[h], si)

    def _wait_idx():
        pltpu.make_async_copy(rows_hbm.at[pl.ds(0, CH)], ri.at[0], si).wait()
        pltpu.make_async_copy(cols_hbm.at[pl.ds(0, CH)], ci.at[0], si).wait()

    def _gissue(h, k, slot):
        pltpu.async_copy(value_hbm.at[ci.at[h, pl.ds(k * SW, SW)]],
                         slots[slot][0], slots[slot][1])

    base = wid * NCH * CH
    _load_idx(base, 0)

    def _chunk(ch, _):
        _wait_idx()
        h = ch % 2

        @pl.when(ch > 0)
        def _drain_prev():
            ix0 = ri.at[h, pl.ds(0, SW)]
            for vb_t, _sg_t, ss_t in slots:
                pltpu.make_async_copy(vb_t, out_sh.at[ix0], ss_t).wait()

        @pl.when(ch + 1 < NCH)
        def _pref():
            _load_idx(base + (ch + 1) * CH, (ch + 1) % 2)

        _gissue(h, 0, 0)
        _gissue(h, 1, 1)

        def _step(k, _):
            rix = ri.at[h, pl.ds(k * SW, SW)]
            for l in range(SW // 16):
                rv = ri[h, pl.ds(k * SW + l * 16, 16)]
                cv = ci[h, pl.ds(k * SW + l * 16, 16)]
                p = _edge_p(s1_v, s2_v, rv, cv)
                plsc.addupdate_scatter(
                    denom_v,
                    [lax.shift_right_logical(rv, 7),
                     lax.bitwise_and(rv, 127)],
                    p)
                p_v[pl.ds(l * 16, 16)] = p

            for sl in range(NS):
                vb_c, sg_c, ss_c = slots[sl]
                nsl = (sl + 2) % NS
                vb_n, sg_n, ss_n = slots[nsl]

                @pl.when(k % NS == sl)
                def _proc():
                    pltpu.make_async_copy(
                        value_hbm.at[rix], vb_c, sg_c).wait()

                    def _scale(jh, _):
                        j2 = jh * 2
                        aj = plsc.load_gather(
                            p_v, [jnp.full((16,), j2, jnp.int32)])
                        bj = plsc.load_gather(
                            p_v, [jnp.full((16,), j2 + 1, jnp.int32)])
                        for l in range(8):
                            vb_c[j2, pl.ds(l * 16, 16)] = (
                                vb_c[j2, pl.ds(l * 16, 16)] * aj)
                        for l in range(8):
                            vb_c[j2 + 1, pl.ds(l * 16, 16)] = (
                                vb_c[j2 + 1, pl.ds(l * 16, 16)] * bj)
                        return 0
                    lax.fori_loop(0, SW // 2, _scale, 0)
                    pltpu.async_copy(vb_c, out_sh.at[rix], ss_c, add=True)

                    @pl.when((k >= 1) & (k + 2 < CR))
                    def _w():
                        pltpu.make_async_copy(
                            vb_n, out_sh.at[rix], ss_n).wait()

                    @pl.when(k + 2 < CR)
                    def _g():
                        _gissue(h, k + 2, nsl)
            return 0
        lax.fori_loop(0, CR, _step, 0)
        return 0
    lax.fori_loop(0, NCH, _chunk, 0)

    ix0 = ri.at[(NCH - 1) % 2, pl.ds(0, SW)]
    for vb_t, _sg_t, ss_t in slots:
        pltpu.make_async_copy(vb_t, out_sh.at[ix0], ss_t).wait()
    plsc.subcore_barrier()

    pltpu.sync_copy(out_sh.at[pl.ds(s * NPT, NPT)],
                    part_hbm.at[c, pl.ds(s * NPT, NPT)])
    pltpu.sync_copy(denom_v, dn_hbm.at[c, s])


def _edge_p(s1_v, s2_v, rv, cv):
    v1 = plsc.load_gather(s1_v, [rv])
    v2 = plsc.load_gather(s2_v, [cv])
    e = v1 + v2
    e = jnp.where(e >= 0.0, e, 0.2 * e)
    return jnp.exp(e)


def _sc_edge_kernel(rows1d, cols1d, s1p, s2p, value):
    mesh = plsc.VectorSubcoreMesh(core_axis_name="c", subcore_axis_name="s")
    f = functools.partial(
        pl.kernel,
        mesh=mesh,
        compiler_params=pltpu.CompilerParams(needs_layout_passes=False),
        out_type=[
            jax.ShapeDtypeStruct((NC, NP, D), jnp.float32),
            jax.ShapeDtypeStruct((NC, NT, DR, 128), jnp.float32),
        ],
        scratch_types=[
            pltpu.VMEM((NP,), jnp.float32),
            pltpu.VMEM((NP,), jnp.float32),
            pltpu.VMEM((DR, 128), jnp.float32),
            pltpu.VMEM((2, CH), jnp.int32),
            pltpu.VMEM((2, CH), jnp.int32),
            pltpu.VMEM((SW,), jnp.float32),
            pltpu.VMEM((SW, D), jnp.float32),
            pltpu.VMEM((SW, D), jnp.float32),
            pltpu.VMEM((SW, D), jnp.float32),
            pltpu.SemaphoreType.DMA,
            pltpu.SemaphoreType.DMA,
            pltpu.SemaphoreType.DMA,
            pltpu.SemaphoreType.DMA,
            pltpu.SemaphoreType.DMA,
            pltpu.SemaphoreType.DMA,
            pltpu.SemaphoreType.DMA,
            pltpu.VMEM_SHARED((NP, D), jnp.float32),
        ],
    )(_sc_body)
    return f(rows1d, cols1d, s1p, s2p, value)


def kernel(x, edge_index, W_map, a1, b1, a2, b2, kernel, bias):
    a12 = jnp.concatenate([a1, a2], axis=1)
    b12 = jnp.stack([b1[0], b2[0]]).reshape(1, 2)
    value, s12 = _tc_precompute(x, W_map, a12, b12, kernel)

    npad = EP - E
    ar = jnp.arange(npad, dtype=jnp.int32)
    prow = N + (ar % (NP - N))
    pcol = ar % 9973
    rows = jnp.concatenate([edge_index[0], prow])
    cols = jnp.concatenate([edge_index[1], pcol])

    zpad = jnp.zeros((NP - N,), jnp.float32)
    s1p = jnp.concatenate([s12[:, 0], zpad])
    s2p = jnp.concatenate([s12[:, 1], zpad])

    part, dn = _sc_edge_kernel(rows, cols, s1p, s2p, value)
    dnT = jnp.transpose(dn.reshape(NW, NP))
    return _tc_combine(part, dnT, bias)
